# baseline (speedup 1.0000x reference)
import functools

import jax
import jax.numpy as jnp
from jax import lax
from jax.experimental import pallas as pl
from jax.experimental.pallas import tpu as pltpu
from jax.experimental.pallas import tpu_sc as plsc

H = 128
W = 128
P = H * W
C = 256
CK = 64
NCK = C // CK
PAD = P
WS = 1.0 / P
WR = 2.0 / P
NW = 32
PPT = P // NW
NBD = 36
D2B = 48
NGB = D2B + NBD + 2
OFFS = [(dy, dx) for dy in (-1, 0, 1) for dx in (-1, 0, 1)]

_mesh = plsc.VectorSubcoreMesh(core_axis_name="c", subcore_axis_name="s")


@functools.partial(
    pl.kernel,
    mesh=_mesh,
    compiler_params=pltpu.CompilerParams(use_tc_tiling_on_sc=False),
    out_type=[
        jax.ShapeDtypeStruct((NCK, 2, P, CK), jnp.float32),
        jax.ShapeDtypeStruct((NCK, NW, PPT, CK), jnp.float32),
        jax.ShapeDtypeStruct((2, P, 16), jnp.float32),
        jax.ShapeDtypeStruct((NW, PPT), jnp.float32),
    ],
    scratch_types=[
        pltpu.VMEM_SHARED((P, CK), jnp.float32),
        pltpu.VMEM_SHARED((P, 16), jnp.float32),
        pltpu.VMEM((PPT + 320,), jnp.int32),
        pltpu.VMEM((PPT + 320,), jnp.int32),
        pltpu.VMEM((PPT,), jnp.int32),
        pltpu.VMEM((PPT,), jnp.int32),
        pltpu.VMEM((NGB * 128,), jnp.int32),
        pltpu.VMEM((NBD, 128), jnp.int32),
        pltpu.VMEM((128,), jnp.int32),
        pltpu.VMEM((128,), jnp.int32),
        pltpu.VMEM((128, CK), jnp.float32),
        pltpu.VMEM((128, CK), jnp.float32),
        pltpu.VMEM((128, CK), jnp.float32),
        pltpu.VMEM((128, 16), jnp.float32),
        pltpu.VMEM((128, 16), jnp.float32),
        pltpu.VMEM((PPT,), jnp.float32),
        pltpu.SemaphoreType.DMA,
        pltpu.SemaphoreType.DMA,
    ],
)
def _sc_vote(ref8, n1y, n1x, n2y, n2x, wtab, acc_out, d1_out, w_out, d1w_out,
             guide_sp, w_sp, n1y_v, n1x_v, n2y_v, n2x_v, gbuf, tbuf,
             wibufA, wibufB, rowsA, rowsB, d1acc, wrowsA, wrowsB, d1w,
             semGA, semGB):
    cid = lax.axis_index("c")
    sid = lax.axis_index("s")
    wid = sid * 2 + cid
    base = wid * PPT

    pltpu.sync_copy(n1y.at[pl.ds(base, PPT + 320)], n1y_v)
    pltpu.sync_copy(n1x.at[pl.ds(base, PPT + 320)], n1x_v)
    pltpu.sync_copy(n2y.at[pl.ds(base + 160, PPT)], n2y_v)
    pltpu.sync_copy(n2x.at[pl.ds(base + 160, PPT)], n2x_v)

    zv16 = jnp.zeros((16,), jnp.float32)
    iot = lax.iota(jnp.int32, 16)

    def zero_d1w(i, carry):
        d1w[pl.ds(i * 16, 16)] = zv16
        return carry

    lax.fori_loop(0, PPT // 16, zero_d1w, 0)

    for oi, (dy, dx) in enumerate(OFFS):
        def build1(j, carry, dy=dy, dx=dx, oi=oi):
            t = base + j * 16 + iot
            ty = lax.shift_right_logical(t, 7)
            tx = lax.bitwise_and(t, W - 1)
            sy = ty - dy
            sx = tx - dx
            loc = j * 16 + 160 - 128 * dy - dx
            my = n1y_v[pl.ds(loc, 16)]
            mx = n1x_v[pl.ds(loc, 16)]
            gy = my + dy
            gx = mx + dx
            m = ((sy >= 0) & (sy < H) & (sx >= 0) & (sx < W)
                 & (gy >= 0) & (gy < H) & (gx >= 0) & (gx < W))
            g = jnp.where(m, gy * W + gx, PAD)
            strip = lax.shift_right_logical(j, 3)
            row = strip * 12 + oi
            col = lax.bitwise_and(j, 7) * 16
            gbuf[pl.ds(row * 128 + col, 16)] = g
            wold = d1w[pl.ds(j * 16, 16)]
            d1w[pl.ds(j * 16, 16)] = wold + jnp.where(m, WS, 0.0)
            return carry

        lax.fori_loop(0, PPT // 16, build1, 0)

    for oi, (dy, dx) in enumerate(OFFS):
        def build2(j, carry, dy=dy, dx=dx, oi=oi):
            r = base + j * 16 + iot
            ry = lax.shift_right_logical(r, 7)
            rx = lax.bitwise_and(r, W - 1)
            my = n2y_v[pl.ds(j * 16, 16)]
            mx = n2x_v[pl.ds(j * 16, 16)]
            ty = my + dy
            tx = mx + dx
            gy = ry + dy
            gx = rx + dx
            m = ((ty >= 0) & (ty < H) & (tx >= 0) & (tx < W)
                 & (gy >= 0) & (gy < H) & (gx >= 0) & (gx < W))
            t = jnp.where(m, ty * W + tx, 0)
            g = jnp.where(m, gy * W + gx, PAD)
            b = oi * 4 + lax.shift_right_logical(j, 3)
            col = lax.bitwise_and(j, 7) * 16
            gbuf[pl.ds((D2B + b) * 128 + col, 16)] = g
            tbuf[b, pl.ds(col, 16)] = t
            return carry

        lax.fori_loop(0, PPT // 16, build2, 0)

    padv = jnp.full((16,), PAD, jnp.int32)
    for row in [s * 12 + o for s in range(4) for o in (9, 10, 11)] + [
            D2B + NBD, D2B + NBD + 1]:
        def fill_pad(i, carry, row=row):
            gbuf[pl.ds(row * 128 + i * 16, 16)] = padv
            return carry

        lax.fori_loop(0, 8, fill_pad, 0)

    def fill_rows_zero(i, carry):
        rowsA[lax.shift_right_logical(i, 2),
              pl.ds(lax.bitwise_and(i, 3) * 16, 16)] = zv16
        return carry

    def fill_wrows_zero(i, carry):
        wrowsA[i, pl.ds(0, 16)] = zv16
        return carry

    def zero_guide():
        lax.fori_loop(0, 512, fill_rows_zero, 0)

        def z(k, carry):
            pltpu.sync_copy(rowsA, guide_sp.at[pl.ds(sid * 1024 + k * 128, 128)])
            return carry

        lax.fori_loop(0, 8, z, 0)

    def zero_w():
        lax.fori_loop(0, 128, fill_wrows_zero, 0)

        def z(k, carry):
            pltpu.sync_copy(wrowsA, w_sp.at[pl.ds(sid * 1024 + k * 128, 128)])
            return carry

        lax.fori_loop(0, 8, z, 0)

    def g_start(b, buf, sem, src):
        pltpu.async_copy(src.at[gbuf.at[pl.ds(b * 128, 128)]], buf, sem)

    def g_wait(b, buf, sem, src):
        pltpu.make_async_copy(
            src.at[gbuf.at[pl.ds(b * 128, 128)]], buf, sem).wait()

    def d2_pipe(src):
        g_start(D2B, rowsA, semGA, src)
        g_start(D2B + 1, rowsB, semGB, src)

        def it(i, carry):
            b0 = D2B + 2 * i
            b1 = b0 + 1
            g_wait(b0, rowsA, semGA, src)
            pltpu.sync_copy(rowsA, guide_sp.at[tbuf.at[2 * i]], add=True)
            g_start(b0 + 2, rowsA, semGA, src)
            g_wait(b1, rowsB, semGB, src)
            pltpu.sync_copy(rowsB, guide_sp.at[tbuf.at[2 * i + 1]], add=True)
            g_start(b1 + 2, rowsB, semGB, src)
            return carry

        lax.fori_loop(0, NBD // 2, it, 0)
        g_wait(D2B + NBD, rowsA, semGA, src)
        g_wait(D2B + NBD + 1, rowsB, semGB, src)

    def _acc_add(buf):
        def addr(r, carry):
            for c4 in range(4):
                d1acc[r, pl.ds(c4 * 16, 16)] = (
                    d1acc[r, pl.ds(c4 * 16, 16)]
                    + buf[r, pl.ds(c4 * 16, 16)])
            return carry

        lax.fori_loop(0, 128, addr, 0, unroll=2)

    def d1_phase(src, ck):
        for s in range(4):
            def zacc(r, carry):
                for c4 in range(4):
                    d1acc[r, pl.ds(c4 * 16, 16)] = zv16
                return carry

            lax.fori_loop(0, 128, zacc, 0, unroll=2)

            r0 = s * 12
            g_start(r0, rowsA, semGA, src)
            g_start(r0 + 1, rowsB, semGB, src)

            def it(i, carry, r0=r0):
                b0 = r0 + 2 * i
                b1 = b0 + 1
                g_wait(b0, rowsA, semGA, src)
                _acc_add(rowsA)
                g_start(b0 + 2, rowsA, semGA, src)
                g_wait(b1, rowsB, semGB, src)
                _acc_add(rowsB)
                g_start(b1 + 2, rowsB, semGB, src)
                return carry

            lax.fori_loop(0, 5, it, 0)
            g_wait(r0 + 10, rowsA, semGA, src)
            g_wait(r0 + 11, rowsB, semGB, src)
            pltpu.sync_copy(d1acc, d1_out.at[ck, wid, pl.ds(s * 128, 128)])

    def wib_build(wib, b):
        def mk(jj, carry):
            g16 = gbuf[pl.ds(b * 128 + jj * 16, 16)]
            wib[pl.ds(jj * 16, 16)] = jnp.where(g16 == PAD, 3, 2)
            return carry

        lax.fori_loop(0, 8, mk, 0)

    def weight_pipe():
        def wg_start(wib, buf, sem):
            pltpu.async_copy(wtab.at[wib], buf, sem)

        def wg_wait(wib, buf, sem):
            pltpu.make_async_copy(wtab.at[wib], buf, sem).wait()

        wib_build(wibufA, D2B)
        wg_start(wibufA, wrowsA, semGA)
        wib_build(wibufB, D2B + 1)
        wg_start(wibufB, wrowsB, semGB)

        def it(i, carry):
            b0 = D2B + 2 * i
            b1 = b0 + 1
            wg_wait(wibufA, wrowsA, semGA)
            pltpu.sync_copy(wrowsA, w_sp.at[tbuf.at[2 * i]], add=True)
            wib_build(wibufA, b0 + 2)
            wg_start(wibufA, wrowsA, semGA)
            wg_wait(wibufB, wrowsB, semGB)
            pltpu.sync_copy(wrowsB, w_sp.at[tbuf.at[2 * i + 1]], add=True)
            wib_build(wibufB, b1 + 2)
            wg_start(wibufB, wrowsB, semGB)
            return carry

        lax.fori_loop(0, NBD // 2, it, 0)
        wg_wait(wibufA, wrowsA, semGA)
        wg_wait(wibufB, wrowsB, semGB)

    def dump_sync(src_sp, dst_hbm, buf):
        def step(k, carry):
            off = sid * 1024 + k * 128
            pltpu.sync_copy(src_sp.at[pl.ds(off, 128)], buf)
            pltpu.sync_copy(buf, dst_hbm.at[pl.ds(off, 128)])
            return carry

        lax.fori_loop(0, 8, step, 0)

    zero_guide()
    zero_w()

    for ck in range(NCK):
        plsc.subcore_barrier()
        if ck == 0:
            weight_pipe()
        d2_pipe(ref8.at[NCK + ck])
        d1_phase(ref8.at[ck], ck)
        plsc.subcore_barrier()
        dump_sync(guide_sp, acc_out.at[ck, cid], rowsA)
        if ck == 0:
            dump_sync(w_sp, w_out.at[cid], wrowsA)
        if ck < NCK - 1:
            zero_guide()

    pltpu.sync_copy(d1w, d1w_out.at[wid])


def _merge_body(acc_ref, d1_ref, w_ref, d1w_ref, out_ref):
    w = w_ref[0, :, 0] + w_ref[1, :, 0] + d1w_ref[...]
    w = jnp.where(w == 0.0, 1.0, w)
    inv = (1.0 / w)[:, None]
    for ck in range(NCK):
        g = acc_ref[ck, 0] + acc_ref[ck, 1] + d1_ref[ck]
        out_ref[:, ck * CK:(ck + 1) * CK] = g * inv


_merge = pl.pallas_call(
    _merge_body,
    grid=(16,),
    in_specs=[
        pl.BlockSpec((NCK, 2, 1024, CK), lambda i: (0, 0, i, 0)),
        pl.BlockSpec((NCK, 1024, CK), lambda i: (0, i, 0)),
        pl.BlockSpec((2, 1024, 16), lambda i: (0, i, 0)),
        pl.BlockSpec((1024,), lambda i: (i,)),
    ],
    out_specs=pl.BlockSpec((1024, C), lambda i: (i, 0)),
    out_shape=jax.ShapeDtypeStruct((P, C), jnp.float32),
)


def kernel(data_A, data_BP, nnf_sr, nnf_rs, curr_layer):
    refT = data_BP[0].reshape(C, P).T
    ref_pad = jnp.concatenate(
        [refT, jnp.zeros((1, C), jnp.float32)], axis=0)
    ref4 = ref_pad.reshape(P + 1, NCK, CK).transpose(1, 0, 2)
    ref8 = jnp.concatenate([WS * ref4, WR * ref4], axis=0)
    zrow = jnp.zeros((160,), jnp.int32)
    n1y = jnp.concatenate(
        [zrow, nnf_sr[..., 0].reshape(P).astype(jnp.int32), zrow])
    n1x = jnp.concatenate(
        [zrow, nnf_sr[..., 1].reshape(P).astype(jnp.int32), zrow])
    n2y = jnp.concatenate(
        [zrow, nnf_rs[..., 0].reshape(P).astype(jnp.int32), zrow])
    n2x = jnp.concatenate(
        [zrow, nnf_rs[..., 1].reshape(P).astype(jnp.int32), zrow])
    wtab = jnp.zeros((4, 16), jnp.float32)
    wtab = wtab.at[0].set(WS).at[2].set(WR)

    acc, d1p, wparts, d1w = _sc_vote(ref8, n1y, n1x, n2y, n2x, wtab)
    guide_flat = _merge(acc, d1p.reshape(NCK, P, CK), wparts,
                        d1w.reshape(P))
    guide = guide_flat.T.reshape(C, H, W)
    return guide, data_A, data_BP


# speedup vs baseline: 2.3448x; 2.3448x over previous
import functools

import jax
import jax.numpy as jnp
from jax import lax
from jax.experimental import pallas as pl
from jax.experimental.pallas import tpu as pltpu
from jax.experimental.pallas import tpu_sc as plsc

H = 128
W = 128
P = H * W
C = 256
CK = 64
NCK = C // CK
PAD = P
WS = 1.0 / P
WR = 2.0 / P
NW = 32
PPT = P // NW
NB = 72
OFFS = [(dy, dx) for dy in (-1, 0, 1) for dx in (-1, 0, 1)]

_mesh = plsc.VectorSubcoreMesh(core_axis_name="c", subcore_axis_name="s")


@functools.partial(
    pl.kernel,
    mesh=_mesh,
    compiler_params=pltpu.CompilerParams(use_tc_tiling_on_sc=False),
    out_type=[
        jax.ShapeDtypeStruct((NCK, 2, P, CK), jnp.float32),
        jax.ShapeDtypeStruct((2, P, 16), jnp.float32),
    ],
    scratch_types=[
        pltpu.VMEM_SHARED((P, CK), jnp.float32),
        pltpu.VMEM_SHARED((P, 16), jnp.float32),
        pltpu.VMEM((PPT,), jnp.int32),
        pltpu.VMEM((PPT,), jnp.int32),
        pltpu.VMEM((PPT,), jnp.int32),
        pltpu.VMEM((PPT,), jnp.int32),
        pltpu.VMEM((NB * 128,), jnp.int32),
        pltpu.VMEM((NB, 128), jnp.int32),
        pltpu.VMEM((128,), jnp.int32),
        pltpu.VMEM((128, CK), jnp.float32),
        pltpu.VMEM((128, CK), jnp.float32),
        pltpu.VMEM((128, 16), jnp.float32),
        pltpu.VMEM((128, 16), jnp.float32),
    ],
)
def _sc_vote(ref8, n1y, n1x, n2y, n2x, zsrc, wtab, acc_out, w_out,
             guide_sp, w_sp, n1y_v, n1x_v, n2y_v, n2x_v, gbuf, tbuf, wibuf,
             rows_v, zrows_v, wrows_v, zw_v):
    cid = lax.axis_index("c")
    sid = lax.axis_index("s")
    wid = sid * 2 + cid
    base = wid * PPT

    pltpu.sync_copy(n1y.at[pl.ds(base, PPT)], n1y_v)
    pltpu.sync_copy(n1x.at[pl.ds(base, PPT)], n1x_v)
    pltpu.sync_copy(n2y.at[pl.ds(base, PPT)], n2y_v)
    pltpu.sync_copy(n2x.at[pl.ds(base, PPT)], n2x_v)
    pltpu.sync_copy(zsrc, zrows_v)

    zv16 = jnp.zeros((16,), jnp.float32)

    def zero_zw(i, carry):
        zw_v[i, pl.ds(0, 16)] = zv16
        return carry

    lax.fori_loop(0, 128, zero_zw, 0)

    iot = lax.iota(jnp.int32, 16)

    for d in range(2):
        ny, nx = (n1y_v, n1x_v) if d == 0 else (n2y_v, n2x_v)
        wv = WS if d == 0 else WR
        for oi, (dy, dx) in enumerate(OFFS):
            q = d * 9 + oi

            def build(j, carry, d=d, dy=dy, dx=dx, q=q, ny=ny, nx=nx, wv=wv):
                p = base + j * 16 + iot
                py = lax.shift_right_logical(p, 7)
                px = lax.bitwise_and(p, W - 1)
                my = ny[pl.ds(j * 16, 16)]
                mx = nx[pl.ds(j * 16, 16)]
                if d == 0:
                    ty = py + dy
                    tx = px + dx
                    gy = my + dy
                    gx = mx + dx
                else:
                    ty = my + dy
                    tx = mx + dx
                    gy = py + dy
                    gx = px + dx
                m = ((ty >= 0) & (ty < H) & (tx >= 0) & (tx < W)
                     & (gy >= 0) & (gy < H) & (gx >= 0) & (gx < W))
                t = jnp.where(m, ty * W + tx, 0)
                g = jnp.where(m, gy * W + gx, PAD)
                gbuf[pl.ds(q * PPT + j * 16, 16)] = g
                b = q * 4 + lax.shift_right_logical(j, 3)
                col = lax.bitwise_and(j, 7) * 16
                tbuf[b, pl.ds(col, 16)] = t
                return carry

            lax.fori_loop(0, PPT // 16, build, 0)

    def zero_slice(k, carry):
        pltpu.sync_copy(zsrc, guide_sp.at[pl.ds(sid * 1024 + k * 128, 128)])
        return carry

    def zero_wslice(k, carry):
        pltpu.sync_copy(zw_v, w_sp.at[pl.ds(sid * 1024 + k * 128, 128)])
        return carry

    lax.fori_loop(0, 8, zero_slice, 0)
    lax.fori_loop(0, 8, zero_wslice, 0)

    for ck in range(NCK):
        plsc.subcore_barrier()
        if ck == 0:
            def wvote(b, carry, dbase=0):
                def mk(jj, c2):
                    g16 = gbuf[pl.ds(b * 128 + jj * 16, 16)]
                    wibuf[pl.ds(jj * 16, 16)] = jnp.where(
                        g16 == PAD, dbase + 1, dbase)
                    return c2

                lax.fori_loop(0, 8, mk, 0)
                pltpu.sync_copy(wtab.at[wibuf], wrows_v)
                pltpu.sync_copy(wrows_v, w_sp.at[tbuf.at[b]], add=True)
                return carry

            lax.fori_loop(0, NB // 2, wvote, 0)
            lax.fori_loop(NB // 2, NB,
                          functools.partial(wvote, dbase=2), 0)

        def vote(b, carry, ck=ck):
            pltpu.sync_copy(ref8.at[ck].at[gbuf.at[pl.ds(b * 128, 128)]], rows_v)
            pltpu.sync_copy(rows_v, guide_sp.at[tbuf.at[b]], add=True)
            return carry

        def vote2(b, carry, ck=ck):
            pltpu.sync_copy(
                ref8.at[NCK + ck].at[gbuf.at[pl.ds(b * 128, 128)]], rows_v)
            pltpu.sync_copy(rows_v, guide_sp.at[tbuf.at[b]], add=True)
            return carry

        lax.fori_loop(0, NB // 2, vote, 0)
        lax.fori_loop(NB // 2, NB, vote2, 0)
        plsc.subcore_barrier()

        def dump(k, carry, ck=ck):
            off = sid * 1024 + k * 128
            pltpu.sync_copy(guide_sp.at[pl.ds(off, 128)],
                            acc_out.at[ck, cid, pl.ds(off, 128)])
            return carry

        lax.fori_loop(0, 8, dump, 0)
        if ck == 0:
            def wdump(k, carry):
                off = sid * 1024 + k * 128
                pltpu.sync_copy(w_sp.at[pl.ds(off, 128)], wrows_v)
                pltpu.sync_copy(wrows_v, w_out.at[cid, pl.ds(off, 128)])
                return carry

            lax.fori_loop(0, 8, wdump, 0)
        if ck < NCK - 1:
            lax.fori_loop(0, 8, zero_slice, 0)


def _merge_body(acc_ref, w_ref, out_ref):
    w = w_ref[0, :, 0] + w_ref[1, :, 0]
    w = jnp.where(w == 0.0, 1.0, w)
    inv = (1.0 / w)[:, None]
    for ck in range(NCK):
        g = acc_ref[ck, 0] + acc_ref[ck, 1]
        out_ref[:, ck * CK:(ck + 1) * CK] = g * inv


_merge = pl.pallas_call(
    _merge_body,
    grid=(16,),
    in_specs=[
        pl.BlockSpec((NCK, 2, 1024, CK), lambda i: (0, 0, i, 0)),
        pl.BlockSpec((2, 1024, 16), lambda i: (0, i, 0)),
    ],
    out_specs=pl.BlockSpec((1024, C), lambda i: (i, 0)),
    out_shape=jax.ShapeDtypeStruct((P, C), jnp.float32),
)


def kernel(data_A, data_BP, nnf_sr, nnf_rs, curr_layer):
    refT = data_BP[0].reshape(C, P).T
    ref_pad = jnp.concatenate(
        [refT, jnp.zeros((1, C), jnp.float32)], axis=0)
    ref4 = ref_pad.reshape(P + 1, NCK, CK).transpose(1, 0, 2)
    ref8 = jnp.concatenate([WS * ref4, WR * ref4], axis=0)
    n1y = nnf_sr[..., 0].reshape(P).astype(jnp.int32)
    n1x = nnf_sr[..., 1].reshape(P).astype(jnp.int32)
    n2y = nnf_rs[..., 0].reshape(P).astype(jnp.int32)
    n2x = nnf_rs[..., 1].reshape(P).astype(jnp.int32)
    zsrc = jnp.zeros((128, CK), jnp.float32)
    wtab = jnp.zeros((4, 16), jnp.float32)
    wtab = wtab.at[0].set(WS).at[2].set(WR)

    acc, wparts = _sc_vote(ref8, n1y, n1x, n2y, n2x, zsrc, wtab)
    guide_flat = _merge(acc, wparts)
    guide = guide_flat.T.reshape(C, H, W)
    return guide, data_A, data_BP


# speedup vs baseline: 2.3730x; 1.0121x over previous
import functools

import jax
import jax.numpy as jnp
from jax import lax
from jax.experimental import pallas as pl
from jax.experimental.pallas import tpu as pltpu
from jax.experimental.pallas import tpu_sc as plsc

H = 128
W = 128
P = H * W
C = 256
CK = 64
NCK = C // CK
PAD = P
WS = 1.0 / P
WR = 2.0 / P
NW = 32
PPT = P // NW
NB = 72
OFFS = [(dy, dx) for dy in (-1, 0, 1) for dx in (-1, 0, 1)]

_mesh = plsc.VectorSubcoreMesh(core_axis_name="c", subcore_axis_name="s")


@functools.partial(
    pl.kernel,
    mesh=_mesh,
    compiler_params=pltpu.CompilerParams(use_tc_tiling_on_sc=False),
    out_type=[
        jax.ShapeDtypeStruct((NCK, 2, P, CK), jnp.float32),
        jax.ShapeDtypeStruct((2, P, 16), jnp.float32),
    ],
    scratch_types=[
        pltpu.VMEM_SHARED((P, CK), jnp.float32),
        pltpu.VMEM_SHARED((P, 16), jnp.float32),
        pltpu.VMEM((PPT,), jnp.int32),
        pltpu.VMEM((PPT,), jnp.int32),
        pltpu.VMEM((PPT,), jnp.int32),
        pltpu.VMEM((PPT,), jnp.int32),
        pltpu.VMEM((NB * 128,), jnp.int32),
        pltpu.VMEM((NB, 128), jnp.int32),
        pltpu.VMEM((128,), jnp.int32),
        pltpu.VMEM((128, CK), jnp.float32),
        pltpu.VMEM((128, CK), jnp.float32),
        pltpu.VMEM((128, 16), jnp.float32),
        pltpu.VMEM((128, 16), jnp.float32),
    ],
)
def _sc_vote(ref8, n1y, n1x, n2y, n2x, zsrc, wtab, acc_out, w_out,
             guide_sp, w_sp, n1y_v, n1x_v, n2y_v, n2x_v, gbuf, tbuf, wibuf,
             rows_v, zrows_v, wrows_v, zw_v):
    cid = lax.axis_index("c")
    sid = lax.axis_index("s")
    wid = sid * 2 + cid
    base = wid * PPT

    pltpu.sync_copy(n1y.at[pl.ds(base, PPT)], n1y_v)
    pltpu.sync_copy(n1x.at[pl.ds(base, PPT)], n1x_v)
    pltpu.sync_copy(n2y.at[pl.ds(base, PPT)], n2y_v)
    pltpu.sync_copy(n2x.at[pl.ds(base, PPT)], n2x_v)
    pltpu.sync_copy(zsrc, zrows_v)

    zv16 = jnp.zeros((16,), jnp.float32)

    def zero_zw(i, carry):
        zw_v[i, pl.ds(0, 16)] = zv16
        return carry

    lax.fori_loop(0, 128, zero_zw, 0)

    iot = lax.iota(jnp.int32, 16)

    for d in range(2):
        ny, nx = (n1y_v, n1x_v) if d == 0 else (n2y_v, n2x_v)
        wv = WS if d == 0 else WR
        for oi, (dy, dx) in enumerate(OFFS):
            q = d * 9 + oi

            def build(j, carry, d=d, dy=dy, dx=dx, q=q, ny=ny, nx=nx, wv=wv):
                p = base + j * 16 + iot
                py = lax.shift_right_logical(p, 7)
                px = lax.bitwise_and(p, W - 1)
                my = ny[pl.ds(j * 16, 16)]
                mx = nx[pl.ds(j * 16, 16)]
                if d == 0:
                    ty = py + dy
                    tx = px + dx
                    gy = my + dy
                    gx = mx + dx
                else:
                    ty = my + dy
                    tx = mx + dx
                    gy = py + dy
                    gx = px + dx
                m = ((ty >= 0) & (ty < H) & (tx >= 0) & (tx < W)
                     & (gy >= 0) & (gy < H) & (gx >= 0) & (gx < W))
                t = jnp.where(m, ty * W + tx, 0)
                g = jnp.where(m, gy * W + gx, PAD)
                gbuf[pl.ds(q * PPT + j * 16, 16)] = g
                b = q * 4 + lax.shift_right_logical(j, 3)
                col = lax.bitwise_and(j, 7) * 16
                tbuf[b, pl.ds(col, 16)] = t
                return carry

            lax.fori_loop(0, PPT // 16, build, 0)

    def zero_slice(k, carry):
        pltpu.sync_copy(zrows_v, guide_sp.at[pl.ds(sid * 1024 + k * 128, 128)])
        return carry

    def zero_wslice(k, carry):
        pltpu.sync_copy(zw_v, w_sp.at[pl.ds(sid * 1024 + k * 128, 128)])
        return carry

    lax.fori_loop(0, 8, zero_slice, 0)
    lax.fori_loop(0, 8, zero_wslice, 0)

    for ck in range(NCK):
        plsc.subcore_barrier()
        if ck == 0:
            def wvote(b, carry, dbase=0):
                def mk(jj, c2):
                    g16 = gbuf[pl.ds(b * 128 + jj * 16, 16)]
                    wibuf[pl.ds(jj * 16, 16)] = jnp.where(
                        g16 == PAD, dbase + 1, dbase)
                    return c2

                lax.fori_loop(0, 8, mk, 0)
                pltpu.sync_copy(wtab.at[wibuf], wrows_v)
                pltpu.sync_copy(wrows_v, w_sp.at[tbuf.at[b]], add=True)
                return carry

            lax.fori_loop(0, NB // 2, wvote, 0)
            lax.fori_loop(NB // 2, NB,
                          functools.partial(wvote, dbase=2), 0)

        def vote(b, carry, ck=ck):
            pltpu.sync_copy(ref8.at[ck].at[gbuf.at[pl.ds(b * 128, 128)]], rows_v)
            pltpu.sync_copy(rows_v, guide_sp.at[tbuf.at[b]], add=True)
            return carry

        def vote2(b, carry, ck=ck):
            pltpu.sync_copy(
                ref8.at[NCK + ck].at[gbuf.at[pl.ds(b * 128, 128)]], rows_v)
            pltpu.sync_copy(rows_v, guide_sp.at[tbuf.at[b]], add=True)
            return carry

        lax.fori_loop(0, NB // 2, vote, 0)
        lax.fori_loop(NB // 2, NB, vote2, 0)
        plsc.subcore_barrier()

        def dump(k, carry, ck=ck):
            off = sid * 1024 + k * 128
            pltpu.sync_copy(guide_sp.at[pl.ds(off, 128)], rows_v)
            pltpu.sync_copy(rows_v, acc_out.at[ck, cid, pl.ds(off, 128)])
            return carry

        lax.fori_loop(0, 8, dump, 0)
        if ck == 0:
            def wdump(k, carry):
                off = sid * 1024 + k * 128
                pltpu.sync_copy(w_sp.at[pl.ds(off, 128)], wrows_v)
                pltpu.sync_copy(wrows_v, w_out.at[cid, pl.ds(off, 128)])
                return carry

            lax.fori_loop(0, 8, wdump, 0)
        if ck < NCK - 1:
            lax.fori_loop(0, 8, zero_slice, 0)


def _merge_body(acc_ref, w_ref, out_ref):
    w = w_ref[0, :, 0] + w_ref[1, :, 0]
    w = jnp.where(w == 0.0, 1.0, w)
    inv = (1.0 / w)[None, :]
    for ck in range(NCK):
        g = acc_ref[ck, 0] + acc_ref[ck, 1]
        out_ref[ck * CK:(ck + 1) * CK, :] = g.T * inv


_merge = pl.pallas_call(
    _merge_body,
    grid=(16,),
    in_specs=[
        pl.BlockSpec((NCK, 2, 1024, CK), lambda i: (0, 0, i, 0)),
        pl.BlockSpec((2, 1024, 16), lambda i: (0, i, 0)),
    ],
    out_specs=pl.BlockSpec((C, 1024), lambda i: (0, i)),
    out_shape=jax.ShapeDtypeStruct((C, P), jnp.float32),
)


def kernel(data_A, data_BP, nnf_sr, nnf_rs, curr_layer):
    refT = data_BP[0].reshape(C, P).T
    ref_pad = jnp.concatenate(
        [refT, jnp.zeros((1, C), jnp.float32)], axis=0)
    ref4 = ref_pad.reshape(P + 1, NCK, CK).transpose(1, 0, 2)
    ref8 = jnp.concatenate([WS * ref4, WR * ref4], axis=0)
    n1y = nnf_sr[..., 0].reshape(P).astype(jnp.int32)
    n1x = nnf_sr[..., 1].reshape(P).astype(jnp.int32)
    n2y = nnf_rs[..., 0].reshape(P).astype(jnp.int32)
    n2x = nnf_rs[..., 1].reshape(P).astype(jnp.int32)
    zsrc = jnp.zeros((128, CK), jnp.float32)
    wtab = jnp.zeros((4, 16), jnp.float32)
    wtab = wtab.at[0].set(WS).at[2].set(WR)

    acc, wparts = _sc_vote(ref8, n1y, n1x, n2y, n2x, zsrc, wtab)
    guide = _merge(acc, wparts).reshape(C, H, W)
    return guide, data_A, data_BP
